# baseline (speedup 1.0000x reference)
import functools

import jax
import jax.numpy as jnp
from jax.experimental import pallas as pl

_N_EXPERTS = 16
_TOP_K = 2

_U32 = jnp.uint32
_KS0 = 0
_KS1 = 42
_KS2 = 42 ^ 0x1BD11BDA
_ROTS = (13, 15, 26, 6, 17, 29, 16, 24)


def _rotl(x, r):
    return (x << _U32(r)) | (x >> _U32(32 - r))


def _threefry_bits(cnt):
    ks = (_U32(_KS0), _U32(_KS1), _U32(_KS2))
    x0 = jnp.zeros_like(cnt) + ks[0]
    x1 = cnt + ks[1]
    for i in range(5):
        rots = _ROTS[:4] if i % 2 == 0 else _ROTS[4:]
        for r in rots:
            x0 = x0 + x1
            x1 = _rotl(x1, r)
            x1 = x0 ^ x1
        x0 = x0 + ks[(i + 1) % 3]
        x1 = x1 + ks[(i + 2) % 3] + _U32(i + 1)
    return x0 ^ x1


def _erfinv_f32(u):
    w = -jnp.log1p(-u * u)
    small = w < 5.0
    ws = w - 2.5
    wl = jnp.sqrt(jnp.where(small, 5.0, w)) - 3.0
    cs = (2.81022636e-08, 3.43273939e-07, -3.5233877e-06, -4.39150654e-06,
          0.00021858087, -0.00125372503, -0.00417768164, 0.246640727,
          1.50140941)
    cl = (-0.000200214257, 0.000100950558, 0.00134934322, -0.00367342844,
          0.00573950773, -0.0076224613, 0.00943887047, 1.00167406,
          2.83297682)
    ps = jnp.float32(cs[0])
    for c in cs[1:]:
        ps = ps * ws + jnp.float32(c)
    plg = jnp.float32(cl[0])
    for c in cl[1:]:
        plg = plg * wl + jnp.float32(c)
    return jnp.where(small, ps, plg) * u


def _eps_block_t(step, rows):
    cnt = (
        _U32(step * rows * _N_EXPERTS)
        + jax.lax.broadcasted_iota(_U32, (_N_EXPERTS, rows), 1) * _U32(_N_EXPERTS)
        + jax.lax.broadcasted_iota(_U32, (_N_EXPERTS, rows), 0)
    )
    bits = _threefry_bits(cnt)
    float_bits = (bits >> _U32(9)) | _U32(0x3F800000)
    f = jax.lax.bitcast_convert_type(float_bits, jnp.float32) - 1.0
    lo = jnp.float32(-0.99999994)
    u = jnp.maximum(lo, f * (1.0 - lo) + lo)
    return jnp.float32(1.41421356) * _erfinv_f32(u)


def _merge(av1, ai1, av2, ai2, bv1, bi1, bv2, bi2):
    c = av1 >= bv1
    v1 = jnp.where(c, av1, bv1)
    i1 = jnp.where(c, ai1, bi1)
    lv = jnp.where(c, bv1, av1)
    li = jnp.where(c, bi1, ai1)
    sv = jnp.where(c, av2, bv2)
    si = jnp.where(c, ai2, bi2)
    c2 = (sv > lv) | (c & (sv == lv))
    v2 = jnp.where(c2, sv, lv)
    i2 = jnp.where(c2, si, li)
    return v1, i1, v2, i2


def _top2_sublanes(noisy_t):
    rows = noisy_t.shape[1]
    v1 = noisy_t
    i1 = jax.lax.broadcasted_iota(jnp.int32, (_N_EXPERTS, rows), 0)
    v2 = jnp.full_like(v1, -jnp.inf)
    i2 = jnp.full_like(i1, _N_EXPERTS)
    g = _N_EXPERTS
    while g > 1:
        h = g // 2
        v1, i1, v2, i2 = _merge(
            v1[:h], i1[:h], v2[:h], i2[:h],
            v1[h:g], i1[h:g], v2[h:g], i2[h:g],
        )
        g = h
    return v1, i1, v2, i2


def _router_kernel(x_ref, w_ref, b_ref, probs_ref, idx_ref):
    rows = x_ref.shape[0]
    acc_t = jax.lax.dot_general(
        w_ref[...], x_ref[...], (((1,), (1,)), ((), ())),
        preferred_element_type=jnp.float32,
    )
    logits_t = acc_t[:_N_EXPERTS, :] + b_ref[:_N_EXPERTS, :]
    noise_t = acc_t[_N_EXPERTS:, :] + b_ref[_N_EXPERTS:, :]
    eps_t = _eps_block_t(pl.program_id(0), rows)
    noisy_t = logits_t + eps_t * jax.nn.softplus(noise_t)

    m1, i1, m2, i2 = _top2_sublanes(noisy_t)

    e = jnp.exp(m2 - m1)
    p1 = 1.0 / (1.0 + e)
    p2 = e / (1.0 + e)
    siota = jax.lax.broadcasted_iota(jnp.int32, (_N_EXPERTS, rows), 0)
    probs_t = jnp.where(siota == i1, p1, jnp.where(siota == i2, p2, 0.0))
    probs_ref[...] = probs_t.T

    kiota = jax.lax.broadcasted_iota(jnp.int32, (_TOP_K, rows), 0)
    idx_t = jnp.where(kiota == 0, i1, i2)
    idx_ref[...] = idx_t.T


@functools.partial(jax.jit, static_argnames=("block_rows",))
def _run(x, w_cat, b_cat, block_rows=512):
    n, d = x.shape
    grid = (n // block_rows,)
    return pl.pallas_call(
        _router_kernel,
        grid=grid,
        in_specs=[
            pl.BlockSpec((block_rows, d), lambda i: (i, 0)),
            pl.BlockSpec((2 * _N_EXPERTS, d), lambda i: (0, 0)),
            pl.BlockSpec((2 * _N_EXPERTS, 1), lambda i: (0, 0)),
        ],
        out_specs=[
            pl.BlockSpec((block_rows, _N_EXPERTS), lambda i: (i, 0)),
            pl.BlockSpec((block_rows, _TOP_K), lambda i: (i, 0)),
        ],
        out_shape=[
            jax.ShapeDtypeStruct((n, _N_EXPERTS), jnp.float32),
            jax.ShapeDtypeStruct((n, _TOP_K), jnp.int32),
        ],
    )(x, w_cat, b_cat)


def kernel(x, route_W, route_b, noise_W, noise_b):
    w_cat = jnp.concatenate([route_W, noise_W], axis=0)
    b_cat = jnp.concatenate([route_b, noise_b], axis=0)[:, None]
    probs, idx = _run(x, w_cat, b_cat)
    return (probs, idx)


# speedup vs baseline: 1.1568x; 1.1568x over previous
import functools

import jax
import jax.numpy as jnp
from jax.experimental import pallas as pl

_N_EXPERTS = 16
_TOP_K = 2

_U32 = jnp.uint32
_KS0 = 0
_KS1 = 42
_KS2 = 42 ^ 0x1BD11BDA
_ROTS = (13, 15, 26, 6, 17, 29, 16, 24)


def _rotl(x, r):
    return (x << _U32(r)) | (x >> _U32(32 - r))


def _threefry_bits(cnt):
    ks = (_U32(_KS0), _U32(_KS1), _U32(_KS2))
    x0 = jnp.zeros_like(cnt) + ks[0]
    x1 = cnt + ks[1]
    for i in range(5):
        rots = _ROTS[:4] if i % 2 == 0 else _ROTS[4:]
        for r in rots:
            x0 = x0 + x1
            x1 = _rotl(x1, r)
            x1 = x0 ^ x1
        x0 = x0 + ks[(i + 1) % 3]
        x1 = x1 + ks[(i + 2) % 3] + _U32(i + 1)
    return x0 ^ x1


def _erfinv_f32(u):
    w = -jnp.log1p(-u * u)
    small = w < 5.0
    ws = w - 2.5
    wl = jnp.sqrt(jnp.where(small, 5.0, w)) - 3.0
    cs = (2.81022636e-08, 3.43273939e-07, -3.5233877e-06, -4.39150654e-06,
          0.00021858087, -0.00125372503, -0.00417768164, 0.246640727,
          1.50140941)
    cl = (-0.000200214257, 0.000100950558, 0.00134934322, -0.00367342844,
          0.00573950773, -0.0076224613, 0.00943887047, 1.00167406,
          2.83297682)
    ps = jnp.float32(cs[0])
    for c in cs[1:]:
        ps = ps * ws + jnp.float32(c)
    plg = jnp.float32(cl[0])
    for c in cl[1:]:
        plg = plg * wl + jnp.float32(c)
    return jnp.where(small, ps, plg) * u


def _eps_block_t(step, rows):
    cnt = (
        _U32(step * rows * _N_EXPERTS)
        + jax.lax.broadcasted_iota(_U32, (_N_EXPERTS, rows), 1) * _U32(_N_EXPERTS)
        + jax.lax.broadcasted_iota(_U32, (_N_EXPERTS, rows), 0)
    )
    bits = _threefry_bits(cnt)
    float_bits = (bits >> _U32(9)) | _U32(0x3F800000)
    f = jax.lax.bitcast_convert_type(float_bits, jnp.float32) - 1.0
    lo = jnp.float32(-0.99999994)
    u = jnp.maximum(lo, f * (1.0 - lo) + lo)
    return jnp.float32(1.41421356) * _erfinv_f32(u)


def _merge(av1, ai1, av2, ai2, bv1, bi1, bv2, bi2):
    c = av1 >= bv1
    v1 = jnp.where(c, av1, bv1)
    i1 = jnp.where(c, ai1, bi1)
    lv = jnp.where(c, bv1, av1)
    li = jnp.where(c, bi1, ai1)
    sv = jnp.where(c, av2, bv2)
    si = jnp.where(c, ai2, bi2)
    c2 = (sv > lv) | (c & (sv == lv))
    v2 = jnp.where(c2, sv, lv)
    i2 = jnp.where(c2, si, li)
    return v1, i1, v2, i2


def _top2_sublanes(noisy_t):
    rows = noisy_t.shape[1]
    v1 = noisy_t
    i1 = jax.lax.broadcasted_iota(jnp.int32, (_N_EXPERTS, rows), 0)
    v2 = jnp.full_like(v1, -jnp.inf)
    i2 = jnp.full_like(i1, _N_EXPERTS)
    g = _N_EXPERTS
    while g > 1:
        h = g // 2
        v1, i1, v2, i2 = _merge(
            v1[:h], i1[:h], v2[:h], i2[:h],
            v1[h:g], i1[h:g], v2[h:g], i2[h:g],
        )
        g = h
    return v1, i1, v2, i2


def _router_kernel(x_ref, w_ref, b_ref, probs_ref, idx_ref):
    rows = x_ref.shape[0]
    acc_t = jax.lax.dot_general(
        w_ref[...], x_ref[...], (((1,), (1,)), ((), ())),
        preferred_element_type=jnp.float32,
    )
    logits_t = acc_t[:_N_EXPERTS, :] + b_ref[:_N_EXPERTS, :]
    noise_t = acc_t[_N_EXPERTS:, :] + b_ref[_N_EXPERTS:, :]
    eps_t = _eps_block_t(pl.program_id(0), rows)
    noisy_t = logits_t + eps_t * jax.nn.softplus(noise_t)

    m1, i1, m2, i2 = _top2_sublanes(noisy_t)

    e = jnp.exp(m2 - m1)
    p1 = 1.0 / (1.0 + e)
    p2 = e / (1.0 + e)
    siota = jax.lax.broadcasted_iota(jnp.int32, (_N_EXPERTS, rows), 0)
    probs_t = jnp.where(siota == i1, p1, jnp.where(siota == i2, p2, 0.0))
    probs_ref[...] = probs_t.T

    kiota = jax.lax.broadcasted_iota(jnp.int32, (_TOP_K, rows), 0)
    idx_t = jnp.where(kiota == 0, i1, i2)
    idx_ref[...] = idx_t.T


@functools.partial(jax.jit, static_argnames=("block_rows",))
def _run(x, w_cat, b_cat, block_rows=2048):
    n, d = x.shape
    grid = (n // block_rows,)
    return pl.pallas_call(
        _router_kernel,
        grid=grid,
        in_specs=[
            pl.BlockSpec((block_rows, d), lambda i: (i, 0)),
            pl.BlockSpec((2 * _N_EXPERTS, d), lambda i: (0, 0)),
            pl.BlockSpec((2 * _N_EXPERTS, 1), lambda i: (0, 0)),
        ],
        out_specs=[
            pl.BlockSpec((block_rows, _N_EXPERTS), lambda i: (i, 0)),
            pl.BlockSpec((block_rows, _TOP_K), lambda i: (i, 0)),
        ],
        out_shape=[
            jax.ShapeDtypeStruct((n, _N_EXPERTS), jnp.float32),
            jax.ShapeDtypeStruct((n, _TOP_K), jnp.int32),
        ],
    )(x, w_cat, b_cat)


def kernel(x, route_W, route_b, noise_W, noise_b):
    w_cat = jnp.concatenate([route_W, noise_W], axis=0)
    b_cat = jnp.concatenate([route_b, noise_b], axis=0)[:, None]
    probs, idx = _run(x, w_cat, b_cat)
    return (probs, idx)
